# Initial kernel scaffold; baseline (speedup 1.0000x reference)
import jax
import jax.numpy as jnp
from jax.experimental import pallas as pl


def kernel(grid_features, vertices, point_features):
    raise NotImplementedError("write your pallas kernel here")


# speedup vs baseline: 3.5094x; 3.5094x over previous
import functools

import jax
import jax.numpy as jnp
from jax import lax
from jax.experimental import pallas as pl
from jax.experimental.pallas import tpu as pltpu
from jax.experimental.pallas import tpu_sc as plsc

_NC = 2
_NS = 16
_NW = _NC * _NS
_L = 16

_G = 128
_C = 16
_B = 400
_NGROUPS = _B // _L
_NROWS = 8 * _B
_NSTREAMS = _NROWS // 128


def _interp_body(table_hbm, xs_hbm, ys_hbm, zs_hbm, out_hbm,
                 xv, yv, zv, idx_v, wt_v, g_v, o_v, sem, csem):
    wid = lax.axis_index("s") * _NC + lax.axis_index("c")
    n_chunks = xs_hbm.shape[0] // _B

    def chunk_body(j, _):
        chunk = wid + _NW * j
        base = chunk * _B

        cp_x = pltpu.async_copy(xs_hbm.at[pl.ds(base, _B)], xv, csem)
        cp_y = pltpu.async_copy(ys_hbm.at[pl.ds(base, _B)], yv, csem)
        cp_z = pltpu.async_copy(zs_hbm.at[pl.ds(base, _B)], zv, csem)
        cp_x.wait()
        cp_y.wait()
        cp_z.wait()

        def group_idx_body(i, _):
            off = i * _L
            x = xv[pl.ds(off, _L)]
            y = yv[pl.ds(off, _L)]
            z = zv[pl.ds(off, _L)]
            half = jnp.float32(0.5 * (_G - 1))
            px = jnp.clip((x + 1.0) * half, 0.0, jnp.float32(_G - 1))
            py = jnp.clip((y + 1.0) * half, 0.0, jnp.float32(_G - 1))
            pz = jnp.clip((z + 1.0) * half, 0.0, jnp.float32(_G - 1))
            ix0 = jnp.minimum(px.astype(jnp.int32), _G - 2)
            iy0 = jnp.minimum(py.astype(jnp.int32), _G - 2)
            iz0 = jnp.minimum(pz.astype(jnp.int32), _G - 2)
            wx = px - ix0.astype(jnp.float32)
            wy = py - iy0.astype(jnp.float32)
            wz = pz - iz0.astype(jnp.float32)
            wx0 = 1.0 - wx
            wy0 = 1.0 - wy
            wz0 = 1.0 - wz

            zy00 = iz0 * (_G * _G) + iy0 * _G
            zy01 = zy00 + _G
            zy10 = zy00 + (_G * _G)
            zy11 = zy10 + _G
            ix1 = ix0 + 1
            idx8 = (zy00 + ix0, zy00 + ix1, zy01 + ix0, zy01 + ix1,
                    zy10 + ix0, zy10 + ix1, zy11 + ix0, zy11 + ix1)

            t00 = wz0 * wy0
            t01 = wz0 * wy
            t10 = wz * wy0
            t11 = wz * wy
            wt8 = (t00 * wx0, t00 * wx, t01 * wx0, t01 * wx,
                   t10 * wx0, t10 * wx, t11 * wx0, t11 * wx)

            for c in range(8):
                idx_v[pl.ds(c * _B + off, _L)] = idx8[c]
                wt_v[c, pl.ds(off, _L)] = wt8[c]
            return 0

        lax.fori_loop(0, _NGROUPS, group_idx_body, 0)

        copies = []
        for s in range(_NSTREAMS):
            copies.append(pltpu.async_copy(
                table_hbm.at[idx_v.at[pl.ds(s * 128, 128)]],
                g_v.at[pl.ds(s * 128, 128)], sem))
        for cp in copies:
            cp.wait()

        def group_sum_body(i, _):
            off = i * _L
            wv = [wt_v[c, pl.ds(off, _L)] for c in range(8)]
            for q in range(_L):
                p = off + q
                acc = g_v[0 * _B + p, :] * wv[0][q]
                for c in range(1, 8):
                    acc = acc + g_v[c * _B + p, :] * wv[c][q]
                o_v[p, :] = acc
            return 0

        lax.fori_loop(0, _NGROUPS, group_sum_body, 0)

        pltpu.async_copy(o_v, out_hbm.at[pl.ds(base, _B)], csem).wait()
        return 0

    my_count = (n_chunks - wid + _NW - 1) // _NW
    lax.fori_loop(0, my_count, chunk_body, 0)


def _make_sc_interp(n_points):
    mesh = plsc.VectorSubcoreMesh(core_axis_name="c", subcore_axis_name="s")
    return functools.partial(
        pl.kernel,
        mesh=mesh,
        out_type=jax.ShapeDtypeStruct((n_points, _C), jnp.float32),
        scratch_types=[
            pltpu.VMEM((_B,), jnp.float32),
            pltpu.VMEM((_B,), jnp.float32),
            pltpu.VMEM((_B,), jnp.float32),
            pltpu.VMEM((_NROWS,), jnp.int32),
            pltpu.VMEM((8, _B), jnp.float32),
            pltpu.VMEM((_NROWS, _C), jnp.float32),
            pltpu.VMEM((_B, _C), jnp.float32),
            pltpu.SemaphoreType.DMA,
            pltpu.SemaphoreType.DMA,
        ],
        compiler_params=pltpu.CompilerParams(use_tc_tiling_on_sc=False),
    )(_interp_body)


def kernel(grid_features, vertices, point_features):
    n = vertices.shape[0]
    table = jnp.transpose(grid_features[0], (1, 2, 3, 0)).reshape(_G * _G * _G, _C)
    xs = vertices[:, 0]
    ys = vertices[:, 1]
    zs = vertices[:, 2]
    sampled = _make_sc_interp(n)(table, xs, ys, zs)
    return jnp.concatenate([point_features, sampled], axis=-1)
